# baseline (speedup 1.0000x reference)
import functools

import jax
import jax.numpy as jnp
import numpy as np
from jax import lax
from jax.experimental import pallas as pl
from jax.experimental.pallas import tpu as pltpu
from jax.experimental.pallas import tpu_sc as plsc

N = 100000
D = 128
V = 95
CHUNK = 400
NCHUNK = N // CHUNK
NW = 32
KMAX = -(-NCHUNK // NW)
REP = 128

_mesh = plsc.VectorSubcoreMesh(core_axis_name="c", subcore_axis_name="s")


@functools.partial(
    pl.kernel,
    mesh=_mesh,
    out_type=jax.ShapeDtypeStruct((N, D), jnp.float32),
    scratch_types=(
        [pltpu.VMEM((CHUNK,), jnp.int32) for _ in range(KMAX)]
        + [pltpu.VMEM((CHUNK, D), jnp.float32) for _ in range(2)]
        + [pltpu.SemaphoreType.DMA, pltpu.SemaphoreType.DMA,
           pltpu.SemaphoreType.DMA]
    ),
)
def _embed_lookup(idx_hbm, table_hbm, out_hbm, *refs):
    idx_v = refs[:KMAX]
    rows_v = refs[KMAX:KMAX + 2]
    sem_i, sem_g, sem_o = refs[KMAX + 2:]
    wid = lax.axis_index("s") * 2 + lax.axis_index("c")

    def cbase(k):
        return pl.multiple_of((wid + NW * k) * CHUNK, 8)

    def idx_copy(k):
        return pltpu.make_async_copy(
            idx_hbm.at[pl.ds(cbase(k), CHUNK)], idx_v[k], sem_i)

    def gather_copy(k, s):
        return pltpu.make_async_copy(
            table_hbm.at[idx_v[k]], rows_v[s], sem_g)

    def out_copy(k, s):
        return pltpu.make_async_copy(
            rows_v[s], out_hbm.at[pl.ds(cbase(k), CHUNK)], sem_o)

    def when_present(k, fn):
        if (k + 1) * NW <= NCHUNK:
            fn()
        else:
            pl.when(wid + NW * k < NCHUNK)(fn)

    def prefetch_idx(k):
        def fn():
            idx_copy(k).start()
        return fn

    def drain_and_flip(k, s):
        def fn():
            gather_copy(k, s).wait()
            out_copy(k, s).start()
        return fn

    def start_chunk(k, s):
        def fn():
            idx_copy(k).wait()
            gather_copy(k, s).start()
        return fn

    def wait_out(k, s):
        def fn():
            out_copy(k, s).wait()
        return fn

    for k in range(KMAX):
        when_present(k, prefetch_idx(k))

    for k in range(KMAX):
        s = k % 2
        if k >= 1:
            when_present(k - 1, drain_and_flip(k - 1, 1 - s))
        if k >= 2:
            when_present(k - 2, wait_out(k - 2, s))
        when_present(k, start_chunk(k, s))

    kl = KMAX - 1
    when_present(kl, drain_and_flip(kl, kl % 2))
    when_present(kl - 1, wait_out(kl - 1, (kl - 1) % 2))
    when_present(kl, wait_out(kl, kl % 2))


_OFFSETS = jnp.asarray((np.arange(N) % REP) * V, dtype=jnp.int32)


def kernel(atomic_num, table):
    idx2 = atomic_num.astype(jnp.int32) + _OFFSETS
    table_rep = jnp.tile(table, (REP, 1))
    return _embed_lookup(idx2, table_rep)


# speedup vs baseline: 1.0696x; 1.0696x over previous
import functools

import jax
import jax.numpy as jnp
import numpy as np
from jax import lax
from jax.experimental import pallas as pl
from jax.experimental.pallas import tpu as pltpu
from jax.experimental.pallas import tpu_sc as plsc

N = 100000
D = 128
V = 95
CHUNK = 400
NCHUNK = N // CHUNK
NW = 32
KMAX = -(-NCHUNK // NW)
REP = 32

_mesh = plsc.VectorSubcoreMesh(core_axis_name="c", subcore_axis_name="s")


@functools.partial(
    pl.kernel,
    mesh=_mesh,
    out_type=jax.ShapeDtypeStruct((N, D), jnp.float32),
    scratch_types=(
        [pltpu.VMEM((CHUNK,), jnp.int32) for _ in range(KMAX)]
        + [pltpu.VMEM((CHUNK, D), jnp.float32) for _ in range(2)]
        + [pltpu.SemaphoreType.DMA, pltpu.SemaphoreType.DMA,
           pltpu.SemaphoreType.DMA]
    ),
)
def _embed_lookup(idx_hbm, table_hbm, out_hbm, *refs):
    idx_v = refs[:KMAX]
    rows_v = refs[KMAX:KMAX + 2]
    sem_i, sem_g, sem_o = refs[KMAX + 2:]
    wid = lax.axis_index("s") * 2 + lax.axis_index("c")

    def cbase(k):
        return pl.multiple_of((wid + NW * k) * CHUNK, 8)

    def idx_copy(k):
        return pltpu.make_async_copy(
            idx_hbm.at[pl.ds(cbase(k), CHUNK)], idx_v[k], sem_i)

    def gather_copy(k, s):
        return pltpu.make_async_copy(
            table_hbm.at[idx_v[k]], rows_v[s], sem_g)

    def out_copy(k, s):
        return pltpu.make_async_copy(
            rows_v[s], out_hbm.at[pl.ds(cbase(k), CHUNK)], sem_o)

    def when_present(k, fn):
        if (k + 1) * NW <= NCHUNK:
            fn()
        else:
            pl.when(wid + NW * k < NCHUNK)(fn)

    def prefetch_idx(k):
        def fn():
            idx_copy(k).start()
        return fn

    def drain_and_flip(k, s):
        def fn():
            gather_copy(k, s).wait()
            out_copy(k, s).start()
        return fn

    def start_chunk(k, s):
        def fn():
            idx_copy(k).wait()
            gather_copy(k, s).start()
        return fn

    def wait_out(k, s):
        def fn():
            out_copy(k, s).wait()
        return fn

    for k in range(KMAX):
        when_present(k, prefetch_idx(k))

    for k in range(KMAX):
        s = k % 2
        if k >= 1:
            when_present(k - 1, drain_and_flip(k - 1, 1 - s))
        if k >= 2:
            when_present(k - 2, wait_out(k - 2, s))
        when_present(k, start_chunk(k, s))

    kl = KMAX - 1
    when_present(kl, drain_and_flip(kl, kl % 2))
    when_present(kl - 1, wait_out(kl - 1, (kl - 1) % 2))
    when_present(kl, wait_out(kl, kl % 2))


_OFFSETS = jnp.asarray((np.arange(N) % REP) * V, dtype=jnp.int32)


def kernel(atomic_num, table):
    idx2 = atomic_num.astype(jnp.int32) + _OFFSETS
    table_rep = jnp.tile(table, (REP, 1))
    return _embed_lookup(idx2, table_rep)


# speedup vs baseline: 1.1323x; 1.0587x over previous
import functools

import jax
import jax.numpy as jnp
from jax import lax
from jax.experimental import pallas as pl
from jax.experimental.pallas import tpu as pltpu
from jax.experimental.pallas import tpu_sc as plsc

N = 100000
D = 128
V = 95
CHUNK = 400
NCHUNK = N // CHUNK
NW = 32
KMAX = -(-NCHUNK // NW)
REP = 64

_mesh = plsc.VectorSubcoreMesh(core_axis_name="c", subcore_axis_name="s")


@functools.partial(
    pl.kernel,
    mesh=_mesh,
    out_type=jax.ShapeDtypeStruct((N, D), jnp.float32),
    scratch_types=(
        [pltpu.VMEM((CHUNK,), jnp.int32) for _ in range(KMAX)]
        + [pltpu.VMEM((CHUNK, D), jnp.float32) for _ in range(2)]
        + [pltpu.SemaphoreType.DMA, pltpu.SemaphoreType.DMA,
           pltpu.SemaphoreType.DMA]
    ),
)
def _embed_lookup(idx_hbm, table_hbm, out_hbm, *refs):
    idx_v = refs[:KMAX]
    rows_v = refs[KMAX:KMAX + 2]
    sem_i, sem_g, sem_o = refs[KMAX + 2:]
    wid = lax.axis_index("s") * 2 + lax.axis_index("c")

    def cbase(k):
        return pl.multiple_of((wid + NW * k) * CHUNK, 8)

    def idx_copy(k):
        return pltpu.make_async_copy(
            idx_hbm.at[pl.ds(cbase(k), CHUNK)], idx_v[k], sem_i)

    def gather_copy(k, s):
        return pltpu.make_async_copy(
            table_hbm.at[idx_v[k]], rows_v[s], sem_g)

    def out_copy(k, s):
        return pltpu.make_async_copy(
            rows_v[s], out_hbm.at[pl.ds(cbase(k), CHUNK)], sem_o)

    def when_present(k, fn):
        if (k + 1) * NW <= NCHUNK:
            fn()
        else:
            pl.when(wid + NW * k < NCHUNK)(fn)

    def prefetch_idx(k):
        def fn():
            idx_copy(k).start()
        return fn

    def drain_and_flip(k, s):
        def fn():
            gather_copy(k, s).wait()
            out_copy(k, s).start()
        return fn

    lanes95 = lax.iota(jnp.int32, 16) * V

    def start_chunk(k, s):
        def fn():
            idx_copy(k).wait()
            iv = idx_v[k]
            c = wid + NW * k

            def off_group(g, carry):
                ph = lax.rem(c + g, 4)
                sl = pl.ds(g * 16, 16)
                iv[sl] = iv[sl] + (lanes95 + ph * (16 * V))
                return carry

            lax.fori_loop(0, CHUNK // 16, off_group, 0)
            gather_copy(k, s).start()
        return fn

    def wait_out(k, s):
        def fn():
            out_copy(k, s).wait()
        return fn

    for k in range(KMAX):
        when_present(k, prefetch_idx(k))

    for k in range(KMAX):
        s = k % 2
        if k >= 1:
            when_present(k - 1, drain_and_flip(k - 1, 1 - s))
        if k >= 2:
            when_present(k - 2, wait_out(k - 2, s))
        when_present(k, start_chunk(k, s))

    kl = KMAX - 1
    when_present(kl, drain_and_flip(kl, kl % 2))
    when_present(kl - 1, wait_out(kl - 1, (kl - 1) % 2))
    when_present(kl, wait_out(kl, kl % 2))


def kernel(atomic_num, table):
    table_rep = jnp.tile(table, (REP, 1))
    return _embed_lookup(atomic_num.astype(jnp.int32), table_rep)
